# baseline (speedup 1.0000x reference)
import functools

import jax
import jax.numpy as jnp
from jax import lax
from jax.experimental import pallas as pl
from jax.experimental.pallas import tpu as pltpu
from jax.experimental.pallas import tpu_sc as plsc

D = 1024
F = 2048
E = 8
T = 2048
NPAIR = 2 * T

TILE = 128
MAX_TILES = NPAIR // TILE + E
MAX_N = MAX_TILES * TILE
QB = 512
NQ = MAX_N // QB

CH = 512
NCH = T // CH

_NC = 2
_NS = 16
_NW = _NC * _NS

GROWS = MAX_N // _NW
GCHUNK = 40
GNB = GROWS // GCHUNK
CTOK = T // _NW
CCHUNK = 16
CNB = CTOK // CCHUNK


def _router_body(x_ref, rw_ref, st_ref, sw_ref, pos_ref, te_ref):
  x = x_ref[...]
  rw = rw_ref[...]
  logits = lax.dot_general(rw, x, (((1,), (1,)), ((), ())),
                           preferred_element_type=jnp.float32)
  m = jnp.max(logits, axis=0, keepdims=True)
  ex = jnp.exp(logits - m)
  probs = ex / jnp.sum(ex, axis=0, keepdims=True)

  row = lax.broadcasted_iota(jnp.int32, (E, T), 0)
  m1 = jnp.max(probs, axis=0, keepdims=True)
  e1 = jnp.min(jnp.where(probs == m1, row, E), axis=0, keepdims=True)
  oh1 = row == e1
  probs2 = jnp.where(oh1, -1.0, probs)
  m2 = jnp.max(probs2, axis=0, keepdims=True)
  e2 = jnp.min(jnp.where(probs2 == m2, row, E), axis=0, keepdims=True)
  oh2 = row == e2
  oh1f = oh1.astype(jnp.float32)
  oh2f = oh2.astype(jnp.float32)

  rc = lax.broadcasted_iota(jnp.int32, (CH, CH), 0)
  cc = lax.broadcasted_iota(jnp.int32, (CH, CH), 1)
  a_strict = (rc < cc).astype(jnp.float32)
  ohcat = jnp.concatenate([oh1f, oh2f], axis=0)
  carry = jnp.zeros((2 * E, 1), jnp.float32)
  chunks = []
  for c in range(NCH):
    blk = ohcat[:, c * CH:(c + 1) * CH]
    within = lax.dot_general(blk, a_strict, (((1,), (0,)), ((), ())),
                             preferred_element_type=jnp.float32)
    chunks.append(within + carry)
    carry = carry + jnp.sum(blk, axis=1, keepdims=True)
  cb = jnp.concatenate(chunks, axis=1)
  cb0 = cb[:E]
  cb1 = cb[E:]
  tot0 = carry[:E]
  tot1 = carry[E:]

  n_i = (tot0 + tot1).astype(jnp.int32)
  pc = ((n_i + (TILE - 1)) // TILE) * TILE
  pcf = pc.astype(jnp.float32)
  re_ = lax.broadcasted_iota(jnp.int32, (E, E), 0)
  ce_ = lax.broadcasted_iota(jnp.int32, (E, E), 1)
  u_strict = (ce_ < re_).astype(jnp.float32)
  off = lax.dot_general(u_strict, pcf, (((1,), (0,)), ((), ())),
                        preferred_element_type=jnp.float32)

  pos0 = jnp.sum(oh1f * (off + cb0), axis=0, keepdims=True)
  pos1 = jnp.sum(oh2f * (off + tot0 + cb1), axis=0, keepdims=True)
  pos_ref[0:1, :] = pos0.astype(jnp.int32)
  pos_ref[1:2, :] = pos1.astype(jnp.int32)

  ti = lax.broadcasted_iota(jnp.int32, (E, MAX_TILES), 1) * TILE
  ge = (ti >= off.astype(jnp.int32)).astype(jnp.int32)
  te_ref[...] = jnp.sum(ge, axis=0, keepdims=True) - 1

  tok = lax.broadcasted_iota(jnp.int32, (1, T), 1).astype(jnp.float32)
  posp = jnp.concatenate([pos0, pos1], axis=1).astype(jnp.int32)
  vals = jnp.concatenate(
      [jnp.concatenate([tok, tok], axis=1),
       jnp.concatenate([m1, m2], axis=1)], axis=0)

  qrow = lax.broadcasted_iota(jnp.int32, (QB, NPAIR), 0)

  def qblock(qi, val):
    qb0 = qi * QB
    mask = (qrow + qb0 == posp).astype(jnp.float32)
    res = lax.dot_general(vals, mask, (((1,), (1,)), ((), ())),
                          preferred_element_type=jnp.float32)
    st_ref[0:1, pl.ds(qb0, QB)] = res[0:1].astype(jnp.int32)
    sw_ref[0:1, pl.ds(qb0, QB)] = res[1:2]
    return val

  lax.fori_loop(0, NQ, qblock, 0)


def _router(x2d, router_w):
  return pl.pallas_call(
      _router_body,
      out_shape=[
          jax.ShapeDtypeStruct((1, MAX_N), jnp.int32),
          jax.ShapeDtypeStruct((1, MAX_N), jnp.float32),
          jax.ShapeDtypeStruct((2, T), jnp.int32),
          jax.ShapeDtypeStruct((1, MAX_TILES), jnp.int32),
      ],
  )(x2d, router_w)


def _gather_body(tok_hbm, x_hbm, out_hbm, idx_v, b0, b1, g0, g1, w0, w1):
  wid = lax.axis_index("s") * _NC + lax.axis_index("c")
  base = wid * GROWS
  pltpu.sync_copy(tok_hbm.at[pl.ds(base, GROWS)], idx_v)
  bufs = (b0, b1)
  gsem = (g0, g1)
  wsem = (w0, w1)
  copies = [None, None]
  for b in range(GNB):
    s = b % 2
    if b >= 2:
      copies[s].wait()
    cp = pltpu.async_copy(
        x_hbm.at[idx_v.at[pl.ds(b * GCHUNK, GCHUNK)]], bufs[s], gsem[s])
    if b >= 1:
      p = (b - 1) % 2
      copies[p] = pltpu.async_copy(
          bufs[p], out_hbm.at[pl.ds(base + (b - 1) * GCHUNK, GCHUNK)],
          wsem[p])
    cp.wait()
  p = (GNB - 1) % 2
  pltpu.async_copy(
      bufs[p], out_hbm.at[pl.ds(base + (GNB - 1) * GCHUNK, GCHUNK)],
      wsem[p]).wait()
  copies[(GNB - 2) % 2].wait()


@functools.lru_cache(maxsize=None)
def _gather_fn():
  return pl.kernel(
      _gather_body,
      out_type=jax.ShapeDtypeStruct((MAX_N, D), jnp.float32),
      mesh=plsc.VectorSubcoreMesh(core_axis_name="c", subcore_axis_name="s"),
      scratch_types=[
          pltpu.VMEM((GROWS,), jnp.int32),
          pltpu.VMEM((GCHUNK, D), jnp.float32),
          pltpu.VMEM((GCHUNK, D), jnp.float32),
          pltpu.SemaphoreType.DMA,
          pltpu.SemaphoreType.DMA,
          pltpu.SemaphoreType.DMA,
          pltpu.SemaphoreType.DMA,
      ],
  )


def _mlp_body(te_ref, xs_ref, w1_ref, w3_ref, w2_ref, sw_ref, ys_ref):
  xb = xs_ref[...]
  g = lax.dot_general(xb, w1_ref[0], (((1,), (1,)), ((), ())),
                      preferred_element_type=jnp.float32)
  u = lax.dot_general(xb, w3_ref[0], (((1,), (1,)), ((), ())),
                      preferred_element_type=jnp.float32)
  h = g * jax.nn.sigmoid(g) * u
  y = lax.dot_general(h, w2_ref[0], (((1,), (1,)), ((), ())),
                      preferred_element_type=jnp.float32)
  ys_ref[...] = y * sw_ref[...]


def _mlp(te, xs, w1, w3, w2, sw):
  grid_spec = pltpu.PrefetchScalarGridSpec(
      num_scalar_prefetch=1,
      grid=(MAX_TILES,),
      in_specs=[
          pl.BlockSpec((TILE, D), lambda i, te: (i, 0)),
          pl.BlockSpec((1, F, D), lambda i, te: (te[i], 0, 0)),
          pl.BlockSpec((1, F, D), lambda i, te: (te[i], 0, 0)),
          pl.BlockSpec((1, D, F), lambda i, te: (te[i], 0, 0)),
          pl.BlockSpec((TILE, 1), lambda i, te: (i, 0)),
      ],
      out_specs=pl.BlockSpec((TILE, D), lambda i, te: (i, 0)),
  )
  return pl.pallas_call(
      _mlp_body,
      grid_spec=grid_spec,
      out_shape=jax.ShapeDtypeStruct((MAX_N, D), jnp.float32),
  )(te, xs, w1, w3, w2, sw)


def _combine_body(pos_hbm, ys_hbm, out_hbm, i0, i1,
                  a0_0, a0_1, a1_0, a1_1, g0, g1, h0, h1, w0, w1):
  wid = lax.axis_index("s") * _NC + lax.axis_index("c")
  base = wid * CTOK
  a0 = (a0_0, a0_1)
  a1 = (a1_0, a1_1)
  gs = (g0, g1)
  hs = (h0, h1)
  ws = (w0, w1)
  pltpu.sync_copy(pos_hbm.at[0, pl.ds(base, CTOK)], i0)
  pltpu.sync_copy(pos_hbm.at[1, pl.ds(base, CTOK)], i1)

  def issue(b):
    s = b % 2
    c0 = pltpu.async_copy(
        ys_hbm.at[i0.at[pl.ds(b * CCHUNK, CCHUNK)]], a0[s], gs[s])
    c1 = pltpu.async_copy(
        ys_hbm.at[i1.at[pl.ds(b * CCHUNK, CCHUNK)]], a1[s], hs[s])
    return c0, c1

  pend = issue(0)
  wcp = [None, None]
  for b in range(CNB):
    s = b % 2
    if b + 1 < CNB:
      s2 = (b + 1) % 2
      if b >= 1:
        wcp[s2].wait()
      nxt = issue(b + 1)
    pend[0].wait()
    pend[1].wait()
    for r in range(CCHUNK):
      def col(j, carry, r=r):
        for q in range(4):
          sl = pl.ds(j * 64 + q * 16, 16)
          a0[s][r, sl] = a0[s][r, sl] + a1[s][r, sl]
        return carry
      lax.fori_loop(0, D // 64, col, 0)
    wcp[s] = pltpu.async_copy(
        a0[s], out_hbm.at[pl.ds(base + b * CCHUNK, CCHUNK)], ws[s])
    if b + 1 < CNB:
      pend = nxt
  wcp[(CNB - 1) % 2].wait()
  wcp[(CNB - 2) % 2].wait()


@functools.lru_cache(maxsize=None)
def _combine_fn():
  return pl.kernel(
      _combine_body,
      out_type=jax.ShapeDtypeStruct((T, D), jnp.float32),
      mesh=plsc.VectorSubcoreMesh(core_axis_name="c", subcore_axis_name="s"),
      scratch_types=[
          pltpu.VMEM((CTOK,), jnp.int32),
          pltpu.VMEM((CTOK,), jnp.int32),
          pltpu.VMEM((CCHUNK, D), jnp.float32),
          pltpu.VMEM((CCHUNK, D), jnp.float32),
          pltpu.VMEM((CCHUNK, D), jnp.float32),
          pltpu.VMEM((CCHUNK, D), jnp.float32),
          pltpu.SemaphoreType.DMA,
          pltpu.SemaphoreType.DMA,
          pltpu.SemaphoreType.DMA,
          pltpu.SemaphoreType.DMA,
          pltpu.SemaphoreType.DMA,
          pltpu.SemaphoreType.DMA,
      ],
  )


def kernel(hidden_states, router_w, w1, w3, w2):
  x2d = hidden_states.reshape(T, D)
  st, sw, pos, te = _router(x2d, router_w)
  xs = _gather_fn()(st.reshape(MAX_N), x2d)
  ys = _mlp(te.reshape(MAX_TILES), xs, w1, w3, w2, sw.reshape(MAX_N, 1))
  out = _combine_fn()(pos, ys)
  return out.reshape(hidden_states.shape)


# speedup vs baseline: 1.0963x; 1.0963x over previous
import functools

import jax
import jax.numpy as jnp
from jax import lax
from jax.experimental import pallas as pl
from jax.experimental.pallas import tpu as pltpu
from jax.experimental.pallas import tpu_sc as plsc

D = 1024
F = 2048
E = 8
T = 2048
NPAIR = 2 * T

TILE = 256
MAX_TILES = NPAIR // TILE + E
MAX_N = MAX_TILES * TILE
QB = 512
NQ = MAX_N // QB

CH = 512
NCH = T // CH

_NC = 2
_NS = 16
_NW = _NC * _NS

GROWS = MAX_N // _NW
GCHUNK = 48
GNB = GROWS // GCHUNK
CTOK = T // _NW
CCHUNK = 16
CNB = CTOK // CCHUNK


def _router_body(x_ref, rw_ref, st_ref, sw_ref, pos_ref, te_ref):
  x = x_ref[...]
  rw = rw_ref[...]
  logits = lax.dot_general(rw, x, (((1,), (1,)), ((), ())),
                           preferred_element_type=jnp.float32)
  m = jnp.max(logits, axis=0, keepdims=True)
  ex = jnp.exp(logits - m)
  probs = ex / jnp.sum(ex, axis=0, keepdims=True)

  row = lax.broadcasted_iota(jnp.int32, (E, T), 0)
  m1 = jnp.max(probs, axis=0, keepdims=True)
  e1 = jnp.min(jnp.where(probs == m1, row, E), axis=0, keepdims=True)
  oh1 = row == e1
  probs2 = jnp.where(oh1, -1.0, probs)
  m2 = jnp.max(probs2, axis=0, keepdims=True)
  e2 = jnp.min(jnp.where(probs2 == m2, row, E), axis=0, keepdims=True)
  oh2 = row == e2
  oh1f = oh1.astype(jnp.float32)
  oh2f = oh2.astype(jnp.float32)

  rc = lax.broadcasted_iota(jnp.int32, (CH, CH), 0)
  cc = lax.broadcasted_iota(jnp.int32, (CH, CH), 1)
  a_strict = (rc < cc).astype(jnp.float32)
  ohcat = jnp.concatenate([oh1f, oh2f], axis=0)
  carry = jnp.zeros((2 * E, 1), jnp.float32)
  chunks = []
  for c in range(NCH):
    blk = ohcat[:, c * CH:(c + 1) * CH]
    within = lax.dot_general(blk, a_strict, (((1,), (0,)), ((), ())),
                             preferred_element_type=jnp.float32)
    chunks.append(within + carry)
    carry = carry + jnp.sum(blk, axis=1, keepdims=True)
  cb = jnp.concatenate(chunks, axis=1)
  cb0 = cb[:E]
  cb1 = cb[E:]
  tot0 = carry[:E]
  tot1 = carry[E:]

  n_i = (tot0 + tot1).astype(jnp.int32)
  pc = ((n_i + (TILE - 1)) // TILE) * TILE
  pcf = pc.astype(jnp.float32)
  re_ = lax.broadcasted_iota(jnp.int32, (E, E), 0)
  ce_ = lax.broadcasted_iota(jnp.int32, (E, E), 1)
  u_strict = (ce_ < re_).astype(jnp.float32)
  off = lax.dot_general(u_strict, pcf, (((1,), (0,)), ((), ())),
                        preferred_element_type=jnp.float32)

  pos0 = jnp.sum(oh1f * (off + cb0), axis=0, keepdims=True)
  pos1 = jnp.sum(oh2f * (off + tot0 + cb1), axis=0, keepdims=True)
  pos_ref[0:1, :] = pos0.astype(jnp.int32)
  pos_ref[1:2, :] = pos1.astype(jnp.int32)

  ti = lax.broadcasted_iota(jnp.int32, (E, MAX_TILES), 1) * TILE
  ge = (ti >= off.astype(jnp.int32)).astype(jnp.int32)
  te_ref[...] = jnp.sum(ge, axis=0, keepdims=True) - 1

  tok = lax.broadcasted_iota(jnp.int32, (1, T), 1).astype(jnp.float32)
  posp = jnp.concatenate([pos0, pos1], axis=1).astype(jnp.int32)
  vals = jnp.concatenate(
      [jnp.concatenate([tok, tok], axis=1),
       jnp.concatenate([m1, m2], axis=1)], axis=0)

  qrow = lax.broadcasted_iota(jnp.int32, (QB, NPAIR), 0)

  def qblock(qi, val):
    qb0 = qi * QB
    mask = (qrow + qb0 == posp).astype(jnp.float32)
    res = lax.dot_general(vals, mask, (((1,), (1,)), ((), ())),
                          preferred_element_type=jnp.float32)
    st_ref[0:1, pl.ds(qb0, QB)] = res[0:1].astype(jnp.int32)
    sw_ref[0:1, pl.ds(qb0, QB)] = res[1:2]
    return val

  lax.fori_loop(0, NQ, qblock, 0)


def _router(x2d, router_w):
  return pl.pallas_call(
      _router_body,
      out_shape=[
          jax.ShapeDtypeStruct((1, MAX_N), jnp.int32),
          jax.ShapeDtypeStruct((1, MAX_N), jnp.float32),
          jax.ShapeDtypeStruct((2, T), jnp.int32),
          jax.ShapeDtypeStruct((1, MAX_TILES), jnp.int32),
      ],
  )(x2d, router_w)


def _gather_body(tok_hbm, x_hbm, out_hbm, idx_v, b0, b1, g0, g1, w0, w1):
  wid = lax.axis_index("s") * _NC + lax.axis_index("c")
  base = wid * GROWS
  pltpu.sync_copy(tok_hbm.at[pl.ds(base, GROWS)], idx_v)
  bufs = (b0, b1)
  gsem = (g0, g1)
  wsem = (w0, w1)
  copies = [None, None]
  for b in range(GNB):
    s = b % 2
    if b >= 2:
      copies[s].wait()
    cp = pltpu.async_copy(
        x_hbm.at[idx_v.at[pl.ds(b * GCHUNK, GCHUNK)]], bufs[s], gsem[s])
    if b >= 1:
      p = (b - 1) % 2
      copies[p] = pltpu.async_copy(
          bufs[p], out_hbm.at[pl.ds(base + (b - 1) * GCHUNK, GCHUNK)],
          wsem[p])
    cp.wait()
  p = (GNB - 1) % 2
  pltpu.async_copy(
      bufs[p], out_hbm.at[pl.ds(base + (GNB - 1) * GCHUNK, GCHUNK)],
      wsem[p]).wait()
  copies[(GNB - 2) % 2].wait()


@functools.lru_cache(maxsize=None)
def _gather_fn():
  return pl.kernel(
      _gather_body,
      out_type=jax.ShapeDtypeStruct((MAX_N, D), jnp.float32),
      mesh=plsc.VectorSubcoreMesh(core_axis_name="c", subcore_axis_name="s"),
      scratch_types=[
          pltpu.VMEM((GROWS,), jnp.int32),
          pltpu.VMEM((GCHUNK, D), jnp.float32),
          pltpu.VMEM((GCHUNK, D), jnp.float32),
          pltpu.SemaphoreType.DMA,
          pltpu.SemaphoreType.DMA,
          pltpu.SemaphoreType.DMA,
          pltpu.SemaphoreType.DMA,
      ],
  )


def _mlp_body(te_ref, xs_ref, w1_ref, w3_ref, w2_ref, sw_ref, ys_ref):
  xb = xs_ref[...].astype(jnp.bfloat16)
  w1b = w1_ref[0].astype(jnp.bfloat16)
  w3b = w3_ref[0].astype(jnp.bfloat16)
  w2b = w2_ref[0].astype(jnp.bfloat16)
  g = lax.dot_general(xb, w1b, (((1,), (1,)), ((), ())),
                      preferred_element_type=jnp.float32)
  u = lax.dot_general(xb, w3b, (((1,), (1,)), ((), ())),
                      preferred_element_type=jnp.float32)
  h = (g * jax.nn.sigmoid(g) * u).astype(jnp.bfloat16)
  y = lax.dot_general(h, w2b, (((1,), (1,)), ((), ())),
                      preferred_element_type=jnp.float32)
  ys_ref[...] = y * sw_ref[...]


def _mlp(te, xs, w1, w3, w2, sw):
  grid_spec = pltpu.PrefetchScalarGridSpec(
      num_scalar_prefetch=1,
      grid=(MAX_TILES,),
      in_specs=[
          pl.BlockSpec((TILE, D), lambda i, te: (i, 0)),
          pl.BlockSpec((1, F, D), lambda i, te: (te[i], 0, 0)),
          pl.BlockSpec((1, F, D), lambda i, te: (te[i], 0, 0)),
          pl.BlockSpec((1, D, F), lambda i, te: (te[i], 0, 0)),
          pl.BlockSpec((TILE, 1), lambda i, te: (i, 0)),
      ],
      out_specs=pl.BlockSpec((TILE, D), lambda i, te: (i, 0)),
  )
  return pl.pallas_call(
      _mlp_body,
      grid_spec=grid_spec,
      out_shape=jax.ShapeDtypeStruct((MAX_N, D), jnp.float32),
  )(te, xs, w1, w3, w2, sw)


def _combine_body(pos_hbm, ys_hbm, out_hbm, i0, i1,
                  a0_0, a0_1, a1_0, a1_1, g0, g1, h0, h1, w0, w1):
  wid = lax.axis_index("s") * _NC + lax.axis_index("c")
  base = wid * CTOK
  a0 = (a0_0, a0_1)
  a1 = (a1_0, a1_1)
  gs = (g0, g1)
  hs = (h0, h1)
  ws = (w0, w1)
  pltpu.sync_copy(pos_hbm.at[0, pl.ds(base, CTOK)], i0)
  pltpu.sync_copy(pos_hbm.at[1, pl.ds(base, CTOK)], i1)

  def issue(b):
    s = b % 2
    c0 = pltpu.async_copy(
        ys_hbm.at[i0.at[pl.ds(b * CCHUNK, CCHUNK)]], a0[s], gs[s])
    c1 = pltpu.async_copy(
        ys_hbm.at[i1.at[pl.ds(b * CCHUNK, CCHUNK)]], a1[s], hs[s])
    return c0, c1

  pend = issue(0)
  wcp = [None, None]
  for b in range(CNB):
    s = b % 2
    if b + 1 < CNB:
      s2 = (b + 1) % 2
      if b >= 1:
        wcp[s2].wait()
      nxt = issue(b + 1)
    pend[0].wait()
    pend[1].wait()
    for r in range(CCHUNK):
      def col(j, carry, r=r):
        for q in range(4):
          sl = pl.ds(j * 64 + q * 16, 16)
          a0[s][r, sl] = a0[s][r, sl] + a1[s][r, sl]
        return carry
      lax.fori_loop(0, D // 64, col, 0)
    wcp[s] = pltpu.async_copy(
        a0[s], out_hbm.at[pl.ds(base + b * CCHUNK, CCHUNK)], ws[s])
    if b + 1 < CNB:
      pend = nxt
  wcp[(CNB - 1) % 2].wait()
  wcp[(CNB - 2) % 2].wait()


@functools.lru_cache(maxsize=None)
def _combine_fn():
  return pl.kernel(
      _combine_body,
      out_type=jax.ShapeDtypeStruct((T, D), jnp.float32),
      mesh=plsc.VectorSubcoreMesh(core_axis_name="c", subcore_axis_name="s"),
      scratch_types=[
          pltpu.VMEM((CTOK,), jnp.int32),
          pltpu.VMEM((CTOK,), jnp.int32),
          pltpu.VMEM((CCHUNK, D), jnp.float32),
          pltpu.VMEM((CCHUNK, D), jnp.float32),
          pltpu.VMEM((CCHUNK, D), jnp.float32),
          pltpu.VMEM((CCHUNK, D), jnp.float32),
          pltpu.SemaphoreType.DMA,
          pltpu.SemaphoreType.DMA,
          pltpu.SemaphoreType.DMA,
          pltpu.SemaphoreType.DMA,
          pltpu.SemaphoreType.DMA,
          pltpu.SemaphoreType.DMA,
      ],
  )


def kernel(hidden_states, router_w, w1, w3, w2):
  x2d = hidden_states.reshape(T, D)
  st, sw, pos, te = _router(x2d, router_w)
  xs = _gather_fn()(st.reshape(MAX_N), x2d)
  ys = _mlp(te.reshape(MAX_TILES), xs, w1, w3, w2, sw.reshape(MAX_N, 1))
  out = _combine_fn()(pos, ys)
  return out.reshape(hidden_states.shape)
